# baseline (speedup 1.0000x reference)
import functools

import jax
import jax.numpy as jnp
from jax import lax
from jax.experimental import pallas as pl
from jax.experimental.pallas import tpu as pltpu
from jax.experimental.pallas import tpu_sc as plsc


def _sc_sqrt(a):
    seed = jnp.where(
        a > 1e-2,
        jnp.float32(0.3),
        jnp.where(
            a > 1e-4,
            jnp.float32(0.03),
            jnp.where(a > 1e-6, jnp.float32(3e-3), jnp.float32(2e-4)),
        ),
    )
    x = jnp.broadcast_to(seed, a.shape)
    for _ in range(8):
        x = 0.5 * (x + a / x)
    return x


def _scales_body(t_hbm, ab_hbm, out_hbm, t_v, ab_g, out_v, sem, *, B):
    cid = lax.axis_index("c")
    sid = lax.axis_index("s")

    @pl.when((cid == 0) & (sid == 0))
    def _():
        pltpu.sync_copy(t_hbm, t_v)
        pltpu.async_copy(ab_hbm.at[t_v], ab_g, sem).wait()
        for i in range(B // 16):
            ab = ab_g[pl.ds(i * 16, 16)]
            out_v[0, pl.ds(i * 16, 16)] = _sc_sqrt(ab)
            out_v[1, pl.ds(i * 16, 16)] = _sc_sqrt(1.0 - ab)
        pltpu.sync_copy(out_v, out_hbm)


def _sc_scales(t, alpha_bars):
    B = t.shape[0]
    mesh = plsc.VectorSubcoreMesh(core_axis_name="c", subcore_axis_name="s")
    fn = functools.partial(
        pl.kernel,
        mesh=mesh,
        out_type=jax.ShapeDtypeStruct((2, B), jnp.float32),
        scratch_types=[
            pltpu.VMEM((B,), jnp.int32),
            pltpu.VMEM((B,), jnp.float32),
            pltpu.VMEM((2, B), jnp.float32),
            pltpu.SemaphoreType.DMA,
        ],
    )(functools.partial(_scales_body, B=B))
    return fn(t, alpha_bars)


def _qsample_body(scales_ref, x0_ref, noise_ref, out_ref, *, nb):
    g = pl.program_id(0)
    for j in range(nb):
        b = g * nb + j
        sa = scales_ref[0, b]
        sb = scales_ref[1, b]
        out_ref[j] = sa * x0_ref[j] + sb * noise_ref[j]


@jax.jit
def kernel(x0, t, noise, alpha_bars):
    B, L, D = x0.shape
    scales = _sc_scales(t, alpha_bars)
    NB = 4
    grid = (B // NB,)
    blk = pl.BlockSpec((NB, L, D), lambda g: (g, 0, 0))
    return pl.pallas_call(
        functools.partial(_qsample_body, nb=NB),
        grid=grid,
        in_specs=[
            pl.BlockSpec(memory_space=pltpu.SMEM),
            blk,
            blk,
        ],
        out_specs=blk,
        out_shape=jax.ShapeDtypeStruct((B, L, D), jnp.float32),
        compiler_params=pltpu.CompilerParams(
            dimension_semantics=("parallel",),
        ),
    )(scales, x0, noise)
